# baseline (speedup 1.0000x reference)
import math

import jax
import jax.numpy as jnp
from jax import lax
from jax.experimental import pallas as pl
from jax.experimental.pallas import tpu as pltpu
import jax.experimental.pallas.tpu_sc as plsc

H = 128
HC = 64
B = 256
BD = 272
NC = 2
NS = 16
L = 16
RB = 128
NG = HC // L
NEG = -3.0e38


def _make_sc_body(n):
  nfull = n // RB
  tail = n - nfull * RB
  nblk = nfull + (1 if tail else 0)
  kmax = (nblk + NS - 1) // NS
  assert tail % L == 0

  def _sc_body(h2_hbm, batch_hbm, stats_out,
               rowbuf, idxg, sumacc, sqacc, maxacc, runbuf, prevbuf):
    cid = lax.axis_index("c")
    sid = lax.axis_index("s")
    iota = lax.iota(jnp.int32, L)

    def fill2d(ref, nrows, val):
      v = jnp.full((L,), val, jnp.float32)
      def body(i, _):
        ref[i // 8, pl.ds((i % 8) * L, L)] = v
        return 0
      lax.fori_loop(0, nrows * 8, body, 0)

    fill2d(maxacc, BD, NEG)
    fill2d(sumacc, BD, 0.0)
    fill2d(sqacc, BD, 0.0)

    def flush(pv):
      cs = pl.ds(HC, L)
      sumacc[pv, cs] = sumacc[pv, cs] + runbuf[0]
      for g in range(NG):
        sl = pl.ds(g * L, L)
        sumacc[pv, sl] = sumacc[pv, sl] + runbuf[1 + g]
        sqacc[pv, sl] = sqacc[pv, sl] + runbuf[5 + g]
        maxacc[pv, sl] = jnp.maximum(maxacc[pv, sl], runbuf[9 + g])

    def reset_run():
      z = jnp.zeros((L,), jnp.float32)
      nv = jnp.full((L,), NEG, jnp.float32)
      for i in range(9):
        runbuf[i] = z
      for i in range(9, 13):
        runbuf[i] = nv

    reset_run()
    prevbuf[0] = jnp.int32(-1)

    def blk_body(k, _carry):
      blk = sid + NS * k
      in_range = blk < nblk
      is_last = blk == (nblk - 1) if tail else jnp.bool_(False)

      @pl.when(in_range & jnp.logical_not(is_last))
      def _():
        pltpu.sync_copy(h2_hbm.at[pl.ds(blk * RB, RB)], rowbuf)
        pltpu.sync_copy(batch_hbm.at[pl.ds(blk * RB, RB)],
                        idxg.at[pl.ds(0, RB)])

      if tail:
        @pl.when(is_last)
        def _():
          pltpu.sync_copy(h2_hbm.at[pl.ds(nfull * RB, tail)],
                          rowbuf.at[pl.ds(0, tail)])
          pltpu.sync_copy(batch_hbm.at[pl.ds(nfull * RB, tail)],
                          idxg.at[pl.ds(0, tail)])
          for off in range(tail, RB, L):
            idxg[pl.ds(off, L)] = jnp.full((L,), B, jnp.int32)

      @pl.when(jnp.logical_not(in_range))
      def _():
        def dfill(i, _):
          idxg[pl.ds(i * L, L)] = jnp.full((L,), B, jnp.int32)
          return 0
        lax.fori_loop(0, (RB + L) // L, dfill, 0)

      def chunk_body(c, _):
        idvec = idxg[pl.ds(c * L, L)]
        prev = prevbuf[0]
        fast = (idvec[0] == prev) & (idvec[L - 1] == prev)

        @pl.when(fast)
        def _():
          sums = [runbuf[1 + g] for g in range(NG)]
          sqs = [runbuf[5 + g] for g in range(NG)]
          mxs = [runbuf[9 + g] for g in range(NG)]
          for j in range(L):
            for g in range(NG):
              v = rowbuf[c * L + j, pl.ds(cid * HC + g * L, L)]
              sums[g] = sums[g] + v
              sqs[g] = sqs[g] + v * v
              mxs[g] = jnp.maximum(mxs[g], v)
          runbuf[0] = runbuf[0] + 16.0
          for g in range(NG):
            runbuf[1 + g] = sums[g]
            runbuf[5 + g] = sqs[g]
            runbuf[9 + g] = mxs[g]

        @pl.when(jnp.logical_not(fast))
        def _():
          for j in range(L):
            s = idvec[j]
            pv = prevbuf[0]
            changed = s != pv

            @pl.when((pv >= 0) & changed)
            def _():
              flush(pv)

            @pl.when(changed)
            def _():
              reset_run()
              prevbuf[0] = s

            runbuf[0] = runbuf[0] + 1.0
            for g in range(NG):
              v = rowbuf[c * L + j, pl.ds(cid * HC + g * L, L)]
              runbuf[1 + g] = runbuf[1 + g] + v
              runbuf[5 + g] = runbuf[5 + g] + v * v
              runbuf[9 + g] = jnp.maximum(runbuf[9 + g], v)
        return 0

      return lax.fori_loop(0, RB // L, chunk_body, 0)

    lax.fori_loop(0, kmax, blk_body, 0)

    pvf = prevbuf[0]

    @pl.when(pvf >= 0)
    def _():
      flush(pvf)

    pltpu.sync_copy(sumacc.at[pl.ds(0, B)], stats_out.at[cid, sid, 0])
    pltpu.sync_copy(sqacc.at[pl.ds(0, B)], stats_out.at[cid, sid, 1])
    pltpu.sync_copy(maxacc.at[pl.ds(0, B)], stats_out.at[cid, sid, 2])

  return _sc_body


def _pool_sc(h, batch, interpret=False):
  n = h.shape[0]
  mesh = plsc.VectorSubcoreMesh(core_axis_name="c", subcore_axis_name="s",
                                num_cores=NC, num_subcores=NS)
  f = pl.kernel(
      _make_sc_body(n),
      out_type=[
          jax.ShapeDtypeStruct((NC, NS, 3, B, H), jnp.float32),
      ],
      mesh=mesh,
      interpret=interpret,
      scratch_types=[
          pltpu.VMEM((RB, H), jnp.float32),
          pltpu.VMEM((RB + L,), jnp.int32),
          pltpu.VMEM((BD, H), jnp.float32),
          pltpu.VMEM((BD, H), jnp.float32),
          pltpu.VMEM((BD, H), jnp.float32),
          pltpu.VMEM((13, L), jnp.float32),
          pltpu.SMEM((1,), jnp.int32),
      ],
  )
  return f(h, batch)[0]


def _tc_body(stats_ref, w1_ref, b1_ref, w2_ref, b2_ref, out_ref):
  st = stats_ref[...]
  s0 = jnp.sum(st[0, :, 0], axis=0)
  s1 = jnp.sum(st[1, :, 0], axis=0)
  q0 = jnp.sum(st[0, :, 1], axis=0)
  q1 = jnp.sum(st[1, :, 1], axis=0)
  m0 = jnp.max(st[0, :, 2], axis=0)
  m1 = jnp.max(st[1, :, 2], axis=0)
  ssum = jnp.concatenate([s0[:, :HC], s1[:, :HC]], axis=1)
  ssq = jnp.concatenate([q0[:, :HC], q1[:, :HC]], axis=1)
  smax = jnp.concatenate([m0[:, :HC], m1[:, :HC]], axis=1)
  count = s0[:, HC]
  safe = jnp.maximum(count, 1.0)[:, None]
  mean = ssum / safe
  var = jnp.maximum(ssq / safe - mean * mean, 0.0)
  std = jnp.sqrt(var + 1e-8)
  smax = jnp.where(count[:, None] > 0.0, smax, 0.0)
  g = jnp.concatenate([mean, smax, std], axis=1)
  hid = jax.nn.relu(
      jnp.dot(g, w1_ref[...], preferred_element_type=jnp.float32)
      + b1_ref[...])
  z = jnp.tanh(
      jnp.dot(hid, w2_ref[...], preferred_element_type=jnp.float32)
      + b2_ref[...]) * math.pi
  out_ref[...] = z


def _head_tc(stats, W1, b1, W2, b2, interpret=False):
  w2p = jnp.zeros((32, 128), jnp.float32).at[:, :8].set(W2)
  b2p = jnp.zeros((1, 128), jnp.float32).at[:, :8].set(b2)
  out = pl.pallas_call(
      _tc_body,
      out_shape=jax.ShapeDtypeStruct((B, 128), jnp.float32),
      interpret=interpret,
  )(stats, W1, b1.reshape(1, 32), w2p, b2p)
  return out[:, :8]


def kernel(h, batch, W1, b1, W2, b2):
  stats = _pool_sc(h, batch)
  return _head_tc(stats, W1, b1, W2, b2)
